# baseline (speedup 1.0000x reference)
import functools

import jax
import jax.numpy as jnp
from jax import lax
from jax.experimental import pallas as pl
from jax.experimental.pallas import tpu as pltpu
from jax.experimental.pallas import tpu_sc as plsc

N = 10000
E = 320000
R = 16
G = 256
D = 128
NP = 12288
NR = 10112
EP = 327680
EPT = EP // 32
NCH = EP // 128
CHT = NCH // 32
PKE = NCH + 2
C = NR * R
CPT = C // 16
ART = NR // 16

_mesh = plsc.VectorSubcoreMesh(core_axis_name="c", subcore_axis_name="s")
f32 = jnp.float32
i32 = jnp.int32


@functools.partial(
    pl.kernel,
    out_type=jax.ShapeDtypeStruct((NP, D), f32),
    mesh=_mesh,
    scratch_types=[
        pltpu.VMEM((128,), i32),
        pltpu.VMEM((128, D), f32),
        pltpu.SemaphoreType.DMA,
    ],
)
def _sc_prep_x(nt_hbm, emb_hbm, x_hbm, nidx, xrows, sem):
    c = lax.axis_index("c")
    s = lax.axis_index("s")
    wid = s * 2 + c
    for k in range(3):
        nb = wid * 384 + k * 128
        pltpu.sync_copy(nt_hbm.at[pl.ds(nb, 128)], nidx)
        pltpu.async_copy(emb_hbm.at[nidx], xrows, sem).wait()
        pltpu.sync_copy(xrows, x_hbm.at[pl.ds(nb, 128)])


@functools.partial(
    pl.kernel,
    out_type=(
        jax.ShapeDtypeStruct((PKE, 3, 128), i32),
        jax.ShapeDtypeStruct((2, C), f32),
    ),
    mesh=_mesh,
    scratch_types=[
        [pltpu.VMEM((3, 128), i32)] * 2,
        [pltpu.VMEM((3, 128), i32)] * 2,
        [pltpu.VMEM((128,), i32)] * 2,
        pltpu.VMEM((128,), f32),
        pltpu.VMEM((CPT,), f32),
        pltpu.VMEM_SHARED((C,), f32),
        [pltpu.SemaphoreType.DMA] * 2,
        [pltpu.SemaphoreType.DMA] * 2,
        [pltpu.SemaphoreType.DMA] * 2,
    ],
)
def _sc_prep(epk_hbm,
             pk_hbm, cntp_hbm,
             ebuf, pbuf, cbuf, ones, zb, cnt_sh,
             isem, wsem, csem):
    c = lax.axis_index("c")
    s = lax.axis_index("s")
    wid = s * 2 + c
    cb = wid * CHT

    for j in range(8):
        ones[pl.ds(j * 16, 16)] = jnp.full((16,), 1.0, f32)

    def zinit(j, _):
        zb[pl.ds(j * 16, 16)] = jnp.zeros((16,), f32)
        return 0
    lax.fori_loop(0, CPT // 16, zinit, 0)
    pltpu.sync_copy(zb, cnt_sh.at[pl.ds(s * CPT, CPT)])
    plsc.subcore_barrier()

    def fetch(sl, k):
        pltpu.async_copy(epk_hbm.at[cb + k], ebuf[sl], isem[sl])

    def process(sl, k):
        pltpu.make_async_copy(epk_hbm.at[cb + k], ebuf[sl], isem[sl]).wait()
        for j in range(8):
            sj = pl.ds(j * 16, 16)
            sv = ebuf[sl][0, sj]
            dv = ebuf[sl][1, sj]
            tv = ebuf[sl][2, sj]
            combv = dv * R + tv
            pbuf[sl][0, sj] = tv * NR + sv
            pbuf[sl][1, sj] = combv
            pbuf[sl][2, sj] = dv
            cbuf[sl][sj] = combv
        pltpu.async_copy(pbuf[sl], pk_hbm.at[cb + k], wsem[sl])
        pltpu.async_copy(ones, cnt_sh.at[cbuf[sl]], csem[sl], add=True)

    def drain(sl, k):
        pltpu.make_async_copy(pbuf[sl], pk_hbm.at[cb + k], wsem[sl]).wait()
        pltpu.make_async_copy(ones, cnt_sh.at[cbuf[sl]], csem[sl]).wait()

    fetch(0, 0)
    fetch(1, 1)

    def it(g, _):
        k0 = 2 * g
        process(0, k0)
        process(1, k0 + 1)
        drain(0, k0)
        fetch(0, k0 + 2)
        drain(1, k0 + 1)
        fetch(1, k0 + 3)
        return 0
    lax.fori_loop(0, CHT // 2, it, 0)
    pltpu.make_async_copy(epk_hbm.at[cb + CHT], ebuf[0], isem[0]).wait()
    pltpu.make_async_copy(epk_hbm.at[cb + CHT + 1], ebuf[1], isem[1]).wait()

    @pl.when(wid == 31)
    def _():
        for j in range(8):
            sj = pl.ds(j * 16, 16)
            zi = jnp.zeros((16,), i32)
            pbuf[0][0, sj] = zi
            pbuf[0][1, sj] = zi
            pbuf[0][2, sj] = zi
        pltpu.sync_copy(pbuf[0], pk_hbm.at[NCH])
        pltpu.sync_copy(pbuf[0], pk_hbm.at[NCH + 1])

    plsc.subcore_barrier()
    pltpu.sync_copy(cnt_sh.at[pl.ds(s * CPT, CPT)],
                    cntp_hbm.at[c, pl.ds(s * CPT, CPT)])


@functools.partial(
    pl.kernel,
    out_type=jax.ShapeDtypeStruct((2, NR, D), f32),
    mesh=_mesh,
    scratch_types=[
        [pltpu.VMEM((3, 128), i32)] * 2,
        [pltpu.VMEM((128, D), f32)] * 2,
        [pltpu.VMEM((128,), f32)] * 2,
        pltpu.VMEM((8, D), f32),
        pltpu.VMEM_SHARED((NR, D), f32),
        [pltpu.SemaphoreType.DMA] * 2,
        [pltpu.SemaphoreType.DMA] * 2,
        [pltpu.SemaphoreType.DMA] * 2,
        [pltpu.SemaphoreType.DMA] * 2,
    ],
)
def _sc_layer(hrows_hbm, pk_hbm, norm_hbm,
              aggp_hbm,
              pkb, rows, nrm, zb, agg_sh, isem, gsem, nsem, ssem):
    c = lax.axis_index("c")
    s = lax.axis_index("s")
    wid = s * 2 + c
    cb = wid * CHT

    for i in range(8):
        for j in range(8):
            zb[i, pl.ds(j * 16, 16)] = jnp.zeros((16,), f32)

    def zcopy(i, _):
        pltpu.sync_copy(zb, agg_sh.at[pl.ds(s * ART + i * 8, 8)])
        return 0
    lax.fori_loop(0, ART // 8, zcopy, 0)
    plsc.subcore_barrier()

    def fetch_idx(sl, k):
        pltpu.async_copy(pk_hbm.at[cb + k], pkb[sl], isem[sl])

    def fetch_rows(sl, k):
        pltpu.make_async_copy(pk_hbm.at[cb + k], pkb[sl], isem[sl]).wait()
        pltpu.async_copy(hrows_hbm.at[pkb[sl].at[0]], rows[sl], gsem[sl])
        pltpu.async_copy(norm_hbm.at[pkb[sl].at[1]], nrm[sl], nsem[sl])

    def process(sl, k):
        pltpu.make_async_copy(hrows_hbm.at[pkb[sl].at[0]], rows[sl],
                              gsem[sl]).wait()
        pltpu.make_async_copy(norm_hbm.at[pkb[sl].at[1]], nrm[sl],
                              nsem[sl]).wait()

        def grp(g, _):
            nv = nrm[sl][pl.ds(g * 16, 16)]
            for l in range(16):
                e = g * 16 + l
                sc = jnp.full((16,), nv[l], f32)
                for cc in range(8):
                    csl = pl.ds(cc * 16, 16)
                    rows[sl][e, csl] = rows[sl][e, csl] * sc
            return 0
        lax.fori_loop(0, 8, grp, 0, unroll=2)
        pltpu.async_copy(rows[sl], agg_sh.at[pkb[sl].at[2]], ssem[sl],
                         add=True)

    def drain(sl):
        pltpu.make_async_copy(rows[sl], agg_sh.at[pkb[sl].at[2]],
                              ssem[sl]).wait()

    fetch_idx(0, 0)
    fetch_idx(1, 1)
    fetch_rows(0, 0)
    fetch_rows(1, 1)

    def it(g, _):
        k0 = 2 * g
        process(0, k0)
        drain(0)
        fetch_idx(0, k0 + 2)
        fetch_rows(0, k0 + 2)
        process(1, k0 + 1)
        drain(1)
        fetch_idx(1, k0 + 3)
        fetch_rows(1, k0 + 3)
        return 0
    lax.fori_loop(0, CHT // 2, it, 0)
    for sl in (0, 1):
        pltpu.make_async_copy(hrows_hbm.at[pkb[sl].at[0]], rows[sl],
                              gsem[sl]).wait()
        pltpu.make_async_copy(norm_hbm.at[pkb[sl].at[1]], nrm[sl],
                              nsem[sl]).wait()

    plsc.subcore_barrier()
    pltpu.sync_copy(agg_sh.at[pl.ds(s * ART, ART)],
                    aggp_hbm.at[c, pl.ds(s * ART, ART)])


def _norm_body(cnt_ref, norm_ref):
    cnt = cnt_ref[0] + cnt_ref[1]
    norm_ref[...] = 1.0 / jnp.maximum(cnt, 1.0)


_tc_norm = pl.pallas_call(
    _norm_body,
    out_shape=jax.ShapeDtypeStruct((C // 128, 128), f32),
)


def _hmat_body(x_ref, w_ref, o_ref):
    o_ref[...] = jnp.dot(x_ref[...], w_ref[0], preferred_element_type=f32)


_tc_hmat = pl.pallas_call(
    _hmat_body,
    grid=(R,),
    in_specs=[pl.BlockSpec((NR, D), lambda r: (0, 0)),
              pl.BlockSpec((1, D, D), lambda r: (r, 0, 0))],
    out_specs=pl.BlockSpec((NR, D), lambda r: (r, 0)),
    out_shape=jax.ShapeDtypeStruct((R * NR, D), f32),
)


def _combine_body(x_ref, p0_ref, p1_ref, root_ref, b_ref, h_ref):
    h = p0_ref[...] + p1_ref[...] + b_ref[...]
    h = h + jnp.dot(x_ref[...], root_ref[...], preferred_element_type=f32)
    h_ref[...] = jnp.maximum(h, 0.0)


_tc_combine = pl.pallas_call(
    _combine_body,
    grid=(NR // 128,),
    in_specs=[pl.BlockSpec((128, D), lambda i: (i, 0)),
              pl.BlockSpec((128, D), lambda i: (i, 0)),
              pl.BlockSpec((128, D), lambda i: (i, 0)),
              pl.BlockSpec((D, D), lambda i: (0, 0)),
              pl.BlockSpec((1, D), lambda i: (0, 0))],
    out_specs=pl.BlockSpec((128, D), lambda i: (i, 0)),
    out_shape=jax.ShapeDtypeStruct((NR, D), f32),
)


def _readout_body(p0_ref, p1_ref, h1_ref, root_ref, b_ref, awt_ref, bs_ref,
                  lw_ref, lb_ref, o_ref, h2_sc, s_sc):
    h2 = p0_ref[...] + p1_ref[...] + b_ref[...]
    h2 = h2 + jnp.dot(h1_ref[...], root_ref[...], preferred_element_type=f32)
    h2 = jnp.maximum(h2, 0.0)
    h2_sc[...] = h2

    neg_inf = jnp.float32(-jnp.inf)
    giota = lax.broadcasted_iota(i32, (G, 128), 0)
    col = lax.broadcasted_iota(i32, (1, 128), 1)

    def oh_mask(b):
        bsrow = bs_ref[pl.ds(b, 1), :]
        vmask = (b * 128 + col) < N
        return (giota == bsrow) & vmask

    def pass1(b, m):
        hblk = h2_sc[pl.ds(b * 128, 128), :]
        scb = lax.dot_general(awt_ref[...], hblk, (((1,), (1,)), ((), ())),
                              preferred_element_type=f32)
        s_sc[pl.ds(b, 1), :] = scb
        oh = oh_mask(b)
        vals = jnp.where(oh, jnp.broadcast_to(scb, (G, 128)), neg_inf)
        return jnp.maximum(m, jnp.max(vals, axis=1, keepdims=True))

    m = lax.fori_loop(0, NR // 128, pass1, jnp.full((G, 1), neg_inf, f32))
    m = jnp.where(jnp.isfinite(m), m, 0.0)

    def pass2(b, carry):
        ssum, ge = carry
        scb = s_sc[pl.ds(b, 1), :]
        oh = oh_mask(b)
        mrow = jnp.sum(jnp.where(oh, jnp.broadcast_to(m, (G, 128)), 0.0),
                       axis=0, keepdims=True)
        erow = jnp.where((b * 128 + col) < N, jnp.exp(scb - mrow), 0.0)
        ohw = jnp.where(oh, jnp.broadcast_to(erow, (G, 128)), 0.0)
        hblk = h2_sc[pl.ds(b * 128, 128), :]
        return (ssum + jnp.sum(ohw, axis=1, keepdims=True),
                ge + jnp.dot(ohw, hblk, preferred_element_type=f32))

    ssum, ge = lax.fori_loop(0, NR // 128, pass2,
                             (jnp.zeros((G, 1), f32), jnp.zeros((G, D), f32)))
    ge = ge / jnp.where(ssum > 0.0, ssum, 1.0)
    rtu = jnp.dot(ge, lw_ref[...], preferred_element_type=f32) + lb_ref[...]
    o_ref[...] = jax.nn.sigmoid(rtu)


_tc_readout = pl.pallas_call(
    _readout_body,
    out_shape=jax.ShapeDtypeStruct((G, 1), f32),
    scratch_shapes=[pltpu.VMEM((NR, D), f32),
                    pltpu.VMEM((NR // 128, 128), f32)],
)


def kernel(nodeTypes, edge_index, edge_attr, bs, emb, W1, root1, b1,
           W2, root2, b2, att_w, lin_w, lin_b):
    src = edge_index[0]
    dst = edge_index[1]
    nt_p = jnp.pad(nodeTypes.astype(i32), (0, NP - N))
    pad_i = jnp.arange(EP - E, dtype=i32)
    src_p = jnp.concatenate([src.astype(i32), pad_i % N])
    dst_p = jnp.concatenate([dst.astype(i32), N + pad_i % (NR - N)])
    et_p = jnp.concatenate([edge_attr.astype(i32), pad_i % R])
    epk = jnp.stack([src_p, dst_p, et_p]).reshape(3, NCH, 128)
    epk = jnp.pad(epk.transpose(1, 0, 2), ((0, 2), (0, 0), (0, 0)))
    bs2 = jnp.pad(bs.astype(i32), (0, NR - N)).reshape(NR // 128, 128)

    x_np = _sc_prep_x(nt_p, emb)
    pk, cntp = _sc_prep(epk)
    norm = _tc_norm(cntp.reshape(2, C // 128, 128)).reshape(C)
    h1rows = _tc_hmat(x_np, W1)
    aggp1 = _sc_layer(h1rows, pk, norm)
    h1 = _tc_combine(x_np, aggp1[0], aggp1[1], root1, b1.reshape(1, D))
    aggp2 = _sc_layer(_tc_hmat(h1, W2), pk, norm)
    out = _tc_readout(aggp2[0], aggp2[1], h1, root2, b2.reshape(1, D),
                      att_w.reshape(D, 1).T, bs2, lin_w, lin_b.reshape(1, 1))
    return out


# speedup vs baseline: 1.0180x; 1.0180x over previous
import functools

import jax
import jax.numpy as jnp
from jax import lax
from jax.experimental import pallas as pl
from jax.experimental.pallas import tpu as pltpu
from jax.experimental.pallas import tpu_sc as plsc

N = 10000
E = 320000
R = 16
G = 256
D = 128
NP = 12288
NR = 10112
EP = 327680
EPT = EP // 32
NCH = EP // 128
CHT = NCH // 32
PKE = NCH + 2
C = NR * R
CPT = C // 16
ART = NR // 16

_mesh = plsc.VectorSubcoreMesh(core_axis_name="c", subcore_axis_name="s")
f32 = jnp.float32
i32 = jnp.int32


@functools.partial(
    pl.kernel,
    out_type=jax.ShapeDtypeStruct((NP, D), f32),
    mesh=_mesh,
    scratch_types=[
        pltpu.VMEM((128,), i32),
        pltpu.VMEM((128, D), f32),
        pltpu.SemaphoreType.DMA,
    ],
)
def _sc_prep_x(nt_hbm, emb_hbm, x_hbm, nidx, xrows, sem):
    c = lax.axis_index("c")
    s = lax.axis_index("s")
    wid = s * 2 + c
    for k in range(3):
        nb = wid * 384 + k * 128
        pltpu.sync_copy(nt_hbm.at[pl.ds(nb, 128)], nidx)
        pltpu.async_copy(emb_hbm.at[nidx], xrows, sem).wait()
        pltpu.sync_copy(xrows, x_hbm.at[pl.ds(nb, 128)])


@functools.partial(
    pl.kernel,
    out_type=(
        jax.ShapeDtypeStruct((PKE, 3, 128), i32),
        jax.ShapeDtypeStruct((2, C), f32),
    ),
    mesh=_mesh,
    scratch_types=[
        [pltpu.VMEM((3, 128), i32)] * 2,
        [pltpu.VMEM((3, 128), i32)] * 2,
        [pltpu.VMEM((128,), i32)] * 2,
        pltpu.VMEM((128,), f32),
        pltpu.VMEM((CPT,), f32),
        pltpu.VMEM_SHARED((C,), f32),
        [pltpu.SemaphoreType.DMA] * 2,
        [pltpu.SemaphoreType.DMA] * 2,
        [pltpu.SemaphoreType.DMA] * 2,
    ],
)
def _sc_prep(epk_hbm,
             pk_hbm, cntp_hbm,
             ebuf, pbuf, cbuf, ones, zb, cnt_sh,
             isem, wsem, csem):
    c = lax.axis_index("c")
    s = lax.axis_index("s")
    wid = s * 2 + c
    cb = wid * CHT

    for j in range(8):
        ones[pl.ds(j * 16, 16)] = jnp.full((16,), 1.0, f32)

    def zinit(j, _):
        zb[pl.ds(j * 16, 16)] = jnp.zeros((16,), f32)
        return 0
    lax.fori_loop(0, CPT // 16, zinit, 0)
    pltpu.sync_copy(zb, cnt_sh.at[pl.ds(s * CPT, CPT)])
    plsc.subcore_barrier()

    def fetch(sl, k):
        pltpu.async_copy(epk_hbm.at[cb + k], ebuf[sl], isem[sl])

    def process(sl, k):
        pltpu.make_async_copy(epk_hbm.at[cb + k], ebuf[sl], isem[sl]).wait()
        for j in range(8):
            sj = pl.ds(j * 16, 16)
            sv = ebuf[sl][0, sj]
            dv = ebuf[sl][1, sj]
            tv = ebuf[sl][2, sj]
            combv = dv * R + tv
            pbuf[sl][0, sj] = tv * NR + sv
            pbuf[sl][1, sj] = combv
            pbuf[sl][2, sj] = dv
            cbuf[sl][sj] = combv
        pltpu.async_copy(pbuf[sl], pk_hbm.at[cb + k], wsem[sl])
        pltpu.async_copy(ones, cnt_sh.at[cbuf[sl]], csem[sl], add=True)

    def drain(sl, k):
        pltpu.make_async_copy(pbuf[sl], pk_hbm.at[cb + k], wsem[sl]).wait()
        pltpu.make_async_copy(ones, cnt_sh.at[cbuf[sl]], csem[sl]).wait()

    fetch(0, 0)
    fetch(1, 1)

    def it(g, _):
        k0 = 2 * g
        process(0, k0)
        process(1, k0 + 1)
        drain(0, k0)
        fetch(0, k0 + 2)
        drain(1, k0 + 1)
        fetch(1, k0 + 3)
        return 0
    lax.fori_loop(0, CHT // 2, it, 0)
    pltpu.make_async_copy(epk_hbm.at[cb + CHT], ebuf[0], isem[0]).wait()
    pltpu.make_async_copy(epk_hbm.at[cb + CHT + 1], ebuf[1], isem[1]).wait()

    @pl.when(wid == 31)
    def _():
        for j in range(8):
            sj = pl.ds(j * 16, 16)
            zi = jnp.zeros((16,), i32)
            pbuf[0][0, sj] = zi
            pbuf[0][1, sj] = zi
            pbuf[0][2, sj] = zi
        pltpu.sync_copy(pbuf[0], pk_hbm.at[NCH])
        pltpu.sync_copy(pbuf[0], pk_hbm.at[NCH + 1])

    plsc.subcore_barrier()
    pltpu.sync_copy(cnt_sh.at[pl.ds(s * CPT, CPT)],
                    cntp_hbm.at[c, pl.ds(s * CPT, CPT)])


@functools.partial(
    pl.kernel,
    out_type=jax.ShapeDtypeStruct((2, NR, D), f32),
    mesh=_mesh,
    scratch_types=[
        [pltpu.VMEM((3, 128), i32)] * 2,
        [pltpu.VMEM((128, D), f32)] * 2,
        [pltpu.VMEM((128,), f32)] * 2,
        pltpu.VMEM((120, D), f32),
        pltpu.VMEM_SHARED((NR, D), f32),
        [pltpu.SemaphoreType.DMA] * 2,
        [pltpu.SemaphoreType.DMA] * 2,
        [pltpu.SemaphoreType.DMA] * 2,
        [pltpu.SemaphoreType.DMA] * 2,
    ],
)
def _sc_layer(hrows_hbm, pk_hbm, norm_hbm,
              aggp_hbm,
              pkb, rows, nrm, zb, agg_sh, isem, gsem, nsem, ssem):
    c = lax.axis_index("c")
    s = lax.axis_index("s")
    wid = s * 2 + c
    cb = wid * CHT

    def zfill(i, _):
        for j in range(8):
            zb[i, pl.ds(j * 16, 16)] = jnp.zeros((16,), f32)
        return 0
    lax.fori_loop(0, 120, zfill, 0)
    for i in range(5):
        pltpu.sync_copy(zb, agg_sh.at[pl.ds(s * ART + i * 120, 120)])
    pltpu.sync_copy(zb.at[pl.ds(0, 32)], agg_sh.at[pl.ds(s * ART + 600, 32)])
    plsc.subcore_barrier()

    def fetch_idx(sl, k):
        pltpu.async_copy(pk_hbm.at[cb + k], pkb[sl], isem[sl])

    def fetch_rows(sl, k):
        pltpu.make_async_copy(pk_hbm.at[cb + k], pkb[sl], isem[sl]).wait()
        pltpu.async_copy(hrows_hbm.at[pkb[sl].at[0]], rows[sl], gsem[sl])
        pltpu.async_copy(norm_hbm.at[pkb[sl].at[1]], nrm[sl], nsem[sl])

    def process(sl, k):
        pltpu.make_async_copy(hrows_hbm.at[pkb[sl].at[0]], rows[sl],
                              gsem[sl]).wait()
        pltpu.make_async_copy(norm_hbm.at[pkb[sl].at[1]], nrm[sl],
                              nsem[sl]).wait()

        def grp(g, _):
            nv = nrm[sl][pl.ds(g * 16, 16)]
            for l in range(16):
                e = g * 16 + l
                sc = jnp.full((16,), nv[l], f32)
                for cc in range(8):
                    csl = pl.ds(cc * 16, 16)
                    rows[sl][e, csl] = rows[sl][e, csl] * sc
            return 0
        lax.fori_loop(0, 8, grp, 0)
        pltpu.async_copy(rows[sl], agg_sh.at[pkb[sl].at[2]], ssem[sl],
                         add=True)

    def drain(sl):
        pltpu.make_async_copy(rows[sl], agg_sh.at[pkb[sl].at[2]],
                              ssem[sl]).wait()

    fetch_idx(0, 0)
    fetch_idx(1, 1)
    fetch_rows(0, 0)
    fetch_rows(1, 1)

    def it(g, _):
        k0 = 2 * g
        process(0, k0)
        drain(0)
        fetch_idx(0, k0 + 2)
        fetch_rows(0, k0 + 2)
        process(1, k0 + 1)
        drain(1)
        fetch_idx(1, k0 + 3)
        fetch_rows(1, k0 + 3)
        return 0
    lax.fori_loop(0, CHT // 2, it, 0)
    for sl in (0, 1):
        pltpu.make_async_copy(hrows_hbm.at[pkb[sl].at[0]], rows[sl],
                              gsem[sl]).wait()
        pltpu.make_async_copy(norm_hbm.at[pkb[sl].at[1]], nrm[sl],
                              nsem[sl]).wait()

    plsc.subcore_barrier()
    pltpu.sync_copy(agg_sh.at[pl.ds(s * ART, ART)],
                    aggp_hbm.at[c, pl.ds(s * ART, ART)])


def _norm_body(cnt_ref, norm_ref):
    cnt = cnt_ref[0] + cnt_ref[1]
    norm_ref[...] = 1.0 / jnp.maximum(cnt, 1.0)


_tc_norm = pl.pallas_call(
    _norm_body,
    out_shape=jax.ShapeDtypeStruct((C // 128, 128), f32),
)


def _hmat_body(x_ref, w_ref, o_ref):
    o_ref[...] = jnp.dot(x_ref[...], w_ref[0], preferred_element_type=f32)


_tc_hmat = pl.pallas_call(
    _hmat_body,
    grid=(R,),
    in_specs=[pl.BlockSpec((NR, D), lambda r: (0, 0)),
              pl.BlockSpec((1, D, D), lambda r: (r, 0, 0))],
    out_specs=pl.BlockSpec((NR, D), lambda r: (r, 0)),
    out_shape=jax.ShapeDtypeStruct((R * NR, D), f32),
)


def _combine_body(x_ref, p0_ref, p1_ref, root_ref, b_ref, h_ref):
    h = p0_ref[...] + p1_ref[...] + b_ref[...]
    h = h + jnp.dot(x_ref[...], root_ref[...], preferred_element_type=f32)
    h_ref[...] = jnp.maximum(h, 0.0)


_tc_combine = pl.pallas_call(
    _combine_body,
    grid=(NR // 128,),
    in_specs=[pl.BlockSpec((128, D), lambda i: (i, 0)),
              pl.BlockSpec((128, D), lambda i: (i, 0)),
              pl.BlockSpec((128, D), lambda i: (i, 0)),
              pl.BlockSpec((D, D), lambda i: (0, 0)),
              pl.BlockSpec((1, D), lambda i: (0, 0))],
    out_specs=pl.BlockSpec((128, D), lambda i: (i, 0)),
    out_shape=jax.ShapeDtypeStruct((NR, D), f32),
)


def _readout_body(p0_ref, p1_ref, h1_ref, root_ref, b_ref, awt_ref, bs_ref,
                  lw_ref, lb_ref, o_ref, h2_sc, s_sc):
    h2 = p0_ref[...] + p1_ref[...] + b_ref[...]
    h2 = h2 + jnp.dot(h1_ref[...], root_ref[...], preferred_element_type=f32)
    h2 = jnp.maximum(h2, 0.0)
    h2_sc[...] = h2

    neg_inf = jnp.float32(-jnp.inf)
    giota = lax.broadcasted_iota(i32, (G, 128), 0)
    col = lax.broadcasted_iota(i32, (1, 128), 1)

    def oh_mask(b):
        bsrow = bs_ref[pl.ds(b, 1), :]
        vmask = (b * 128 + col) < N
        return (giota == bsrow) & vmask

    def pass1(b, m):
        hblk = h2_sc[pl.ds(b * 128, 128), :]
        scb = lax.dot_general(awt_ref[...], hblk, (((1,), (1,)), ((), ())),
                              preferred_element_type=f32)
        s_sc[pl.ds(b, 1), :] = scb
        oh = oh_mask(b)
        vals = jnp.where(oh, jnp.broadcast_to(scb, (G, 128)), neg_inf)
        return jnp.maximum(m, jnp.max(vals, axis=1, keepdims=True))

    m = lax.fori_loop(0, NR // 128, pass1, jnp.full((G, 1), neg_inf, f32))
    m = jnp.where(jnp.isfinite(m), m, 0.0)

    def pass2(b, carry):
        ssum, ge = carry
        scb = s_sc[pl.ds(b, 1), :]
        oh = oh_mask(b)
        mrow = jnp.sum(jnp.where(oh, jnp.broadcast_to(m, (G, 128)), 0.0),
                       axis=0, keepdims=True)
        erow = jnp.where((b * 128 + col) < N, jnp.exp(scb - mrow), 0.0)
        ohw = jnp.where(oh, jnp.broadcast_to(erow, (G, 128)), 0.0)
        hblk = h2_sc[pl.ds(b * 128, 128), :]
        return (ssum + jnp.sum(ohw, axis=1, keepdims=True),
                ge + jnp.dot(ohw, hblk, preferred_element_type=f32))

    ssum, ge = lax.fori_loop(0, NR // 128, pass2,
                             (jnp.zeros((G, 1), f32), jnp.zeros((G, D), f32)))
    ge = ge / jnp.where(ssum > 0.0, ssum, 1.0)
    rtu = jnp.dot(ge, lw_ref[...], preferred_element_type=f32) + lb_ref[...]
    o_ref[...] = jax.nn.sigmoid(rtu)


_tc_readout = pl.pallas_call(
    _readout_body,
    out_shape=jax.ShapeDtypeStruct((G, 1), f32),
    scratch_shapes=[pltpu.VMEM((NR, D), f32),
                    pltpu.VMEM((NR // 128, 128), f32)],
)


def kernel(nodeTypes, edge_index, edge_attr, bs, emb, W1, root1, b1,
           W2, root2, b2, att_w, lin_w, lin_b):
    src = edge_index[0]
    dst = edge_index[1]
    nt_p = jnp.pad(nodeTypes.astype(i32), (0, NP - N))
    pad_i = jnp.arange(EP - E, dtype=i32)
    src_p = jnp.concatenate([src.astype(i32), pad_i % N])
    dst_p = jnp.concatenate([dst.astype(i32), N + pad_i % (NR - N)])
    et_p = jnp.concatenate([edge_attr.astype(i32), pad_i % R])
    epk = jnp.stack([src_p, dst_p, et_p]).reshape(3, NCH, 128)
    epk = jnp.pad(epk.transpose(1, 0, 2), ((0, 2), (0, 0), (0, 0)))
    bs2 = jnp.pad(bs.astype(i32), (0, NR - N)).reshape(NR // 128, 128)

    x_np = _sc_prep_x(nt_p, emb)
    pk, cntp = _sc_prep(epk)
    norm = _tc_norm(cntp.reshape(2, C // 128, 128)).reshape(C)
    h1rows = _tc_hmat(x_np, W1)
    aggp1 = _sc_layer(h1rows, pk, norm)
    h1 = _tc_combine(x_np, aggp1[0], aggp1[1], root1, b1.reshape(1, D))
    aggp2 = _sc_layer(_tc_hmat(h1, W2), pk, norm)
    out = _tc_readout(aggp2[0], aggp2[1], h1, root2, b2.reshape(1, D),
                      att_w.reshape(D, 1).T, bs2, lin_w, lin_b.reshape(1, 1))
    return out


# speedup vs baseline: 1.0343x; 1.0160x over previous
import functools

import jax
import jax.numpy as jnp
from jax import lax
from jax.experimental import pallas as pl
from jax.experimental.pallas import tpu as pltpu
from jax.experimental.pallas import tpu_sc as plsc

N = 10000
E = 320000
R = 16
G = 256
D = 128
NP = 12288
NR = 10112
EP = 327680
EPT = EP // 32
NCH = EP // 128
CHT = NCH // 32
PKE = NCH + 4
C = NR * R
CPT = C // 16
ART = NR // 16

_mesh = plsc.VectorSubcoreMesh(core_axis_name="c", subcore_axis_name="s")
f32 = jnp.float32
i32 = jnp.int32


@functools.partial(
    pl.kernel,
    out_type=jax.ShapeDtypeStruct((NP, D), f32),
    mesh=_mesh,
    scratch_types=[
        pltpu.VMEM((128,), i32),
        pltpu.VMEM((128, D), f32),
        pltpu.SemaphoreType.DMA,
    ],
)
def _sc_prep_x(nt_hbm, emb_hbm, x_hbm, nidx, xrows, sem):
    c = lax.axis_index("c")
    s = lax.axis_index("s")
    wid = s * 2 + c
    for k in range(3):
        nb = wid * 384 + k * 128
        pltpu.sync_copy(nt_hbm.at[pl.ds(nb, 128)], nidx)
        pltpu.async_copy(emb_hbm.at[nidx], xrows, sem).wait()
        pltpu.sync_copy(xrows, x_hbm.at[pl.ds(nb, 128)])


@functools.partial(
    pl.kernel,
    out_type=(
        jax.ShapeDtypeStruct((PKE, 3, 128), i32),
        jax.ShapeDtypeStruct((2, C), f32),
    ),
    mesh=_mesh,
    scratch_types=[
        [pltpu.VMEM((2, 3, 128), i32)] * 2,
        [pltpu.VMEM((2, 3, 128), i32)] * 2,
        [pltpu.VMEM((128,), i32)] * 4,
        pltpu.VMEM((128,), f32),
        pltpu.VMEM((CPT,), f32),
        pltpu.VMEM_SHARED((C,), f32),
        [pltpu.SemaphoreType.DMA] * 2,
        [pltpu.SemaphoreType.DMA] * 2,
        [pltpu.SemaphoreType.DMA] * 2,
    ],
)
def _sc_prep(epk_hbm,
             pk_hbm, cntp_hbm,
             ebuf, pbuf, cbuf, ones, zb, cnt_sh,
             isem, wsem, csem):
    c = lax.axis_index("c")
    s = lax.axis_index("s")
    wid = s * 2 + c
    cb = wid * CHT

    for j in range(8):
        ones[pl.ds(j * 16, 16)] = jnp.full((16,), 1.0, f32)

    def zinit(j, _):
        zb[pl.ds(j * 16, 16)] = jnp.zeros((16,), f32)
        return 0
    lax.fori_loop(0, CPT // 16, zinit, 0)
    pltpu.sync_copy(zb, cnt_sh.at[pl.ds(s * CPT, CPT)])
    plsc.subcore_barrier()

    def fetch(sl, k):
        pltpu.async_copy(epk_hbm.at[pl.ds(cb + k, 2)], ebuf[sl], isem[sl])

    def process(sl, k):
        pltpu.make_async_copy(epk_hbm.at[pl.ds(cb + k, 2)], ebuf[sl],
                              isem[sl]).wait()
        for sub in range(2):
            for j in range(8):
                sj = pl.ds(j * 16, 16)
                sv = ebuf[sl][sub, 0, sj]
                dv = ebuf[sl][sub, 1, sj]
                tv = ebuf[sl][sub, 2, sj]
                combv = dv * R + tv
                pbuf[sl][sub, 0, sj] = tv * NR + sv
                pbuf[sl][sub, 1, sj] = combv
                pbuf[sl][sub, 2, sj] = dv
                cbuf[sl * 2 + sub][sj] = combv
        pltpu.async_copy(pbuf[sl], pk_hbm.at[pl.ds(cb + k, 2)], wsem[sl])
        pltpu.async_copy(ones, cnt_sh.at[cbuf[sl * 2]], csem[sl], add=True)
        pltpu.async_copy(ones, cnt_sh.at[cbuf[sl * 2 + 1]], csem[sl],
                         add=True)

    def drain(sl, k):
        pltpu.make_async_copy(pbuf[sl], pk_hbm.at[pl.ds(cb + k, 2)],
                              wsem[sl]).wait()
        pltpu.make_async_copy(ones, cnt_sh.at[cbuf[sl * 2]], csem[sl]).wait()
        pltpu.make_async_copy(ones, cnt_sh.at[cbuf[sl * 2 + 1]],
                              csem[sl]).wait()

    fetch(0, 0)
    fetch(1, 2)

    def it(g, _):
        k0 = 4 * g
        process(0, k0)
        process(1, k0 + 2)
        drain(0, k0)
        fetch(0, k0 + 4)
        drain(1, k0 + 2)
        fetch(1, k0 + 6)
        return 0
    lax.fori_loop(0, CHT // 4, it, 0)
    pltpu.make_async_copy(epk_hbm.at[pl.ds(cb + CHT, 2)], ebuf[0],
                          isem[0]).wait()
    pltpu.make_async_copy(epk_hbm.at[pl.ds(cb + CHT + 2, 2)], ebuf[1],
                          isem[1]).wait()

    @pl.when(wid == 31)
    def _():
        for sub in range(2):
            for j in range(8):
                sj = pl.ds(j * 16, 16)
                zi = jnp.zeros((16,), i32)
                pbuf[0][sub, 0, sj] = zi
                pbuf[0][sub, 1, sj] = zi
                pbuf[0][sub, 2, sj] = zi
        pltpu.sync_copy(pbuf[0], pk_hbm.at[pl.ds(NCH, 2)])
        pltpu.sync_copy(pbuf[0], pk_hbm.at[pl.ds(NCH + 2, 2)])

    plsc.subcore_barrier()
    pltpu.sync_copy(cnt_sh.at[pl.ds(s * CPT, CPT)],
                    cntp_hbm.at[c, pl.ds(s * CPT, CPT)])


@functools.partial(
    pl.kernel,
    out_type=jax.ShapeDtypeStruct((2, NR, D), f32),
    mesh=_mesh,
    scratch_types=[
        [pltpu.VMEM((3, 128), i32)] * 2,
        [pltpu.VMEM((128, D), f32)] * 2,
        [pltpu.VMEM((128,), f32)] * 2,
        pltpu.VMEM((120, D), f32),
        pltpu.VMEM_SHARED((NR, D), f32),
        [pltpu.SemaphoreType.DMA] * 2,
        [pltpu.SemaphoreType.DMA] * 2,
        [pltpu.SemaphoreType.DMA] * 2,
        [pltpu.SemaphoreType.DMA] * 2,
    ],
)
def _sc_layer(hrows_hbm, pk_hbm, norm_hbm,
              aggp_hbm,
              pkb, rows, nrm, zb, agg_sh, isem, gsem, nsem, ssem):
    c = lax.axis_index("c")
    s = lax.axis_index("s")
    wid = s * 2 + c
    cb = wid * CHT

    def zfill(i, _):
        for j in range(8):
            zb[i, pl.ds(j * 16, 16)] = jnp.zeros((16,), f32)
        return 0
    lax.fori_loop(0, 120, zfill, 0)
    for i in range(5):
        pltpu.sync_copy(zb, agg_sh.at[pl.ds(s * ART + i * 120, 120)])
    pltpu.sync_copy(zb.at[pl.ds(0, 32)], agg_sh.at[pl.ds(s * ART + 600, 32)])
    plsc.subcore_barrier()

    def fetch_idx(sl, k):
        pltpu.async_copy(pk_hbm.at[cb + k], pkb[sl], isem[sl])

    def fetch_rows(sl, k):
        pltpu.make_async_copy(pk_hbm.at[cb + k], pkb[sl], isem[sl]).wait()
        pltpu.async_copy(hrows_hbm.at[pkb[sl].at[0]], rows[sl], gsem[sl])
        pltpu.async_copy(norm_hbm.at[pkb[sl].at[1]], nrm[sl], nsem[sl])

    def process(sl, k):
        pltpu.make_async_copy(hrows_hbm.at[pkb[sl].at[0]], rows[sl],
                              gsem[sl]).wait()
        pltpu.make_async_copy(norm_hbm.at[pkb[sl].at[1]], nrm[sl],
                              nsem[sl]).wait()

        def grp(g, _):
            nv = nrm[sl][pl.ds(g * 16, 16)]
            for l in range(16):
                e = g * 16 + l
                sc = jnp.full((16,), nv[l], f32)
                for cc in range(8):
                    csl = pl.ds(cc * 16, 16)
                    rows[sl][e, csl] = rows[sl][e, csl] * sc
            return 0
        lax.fori_loop(0, 8, grp, 0)
        pltpu.async_copy(rows[sl], agg_sh.at[pkb[sl].at[2]], ssem[sl],
                         add=True)

    def drain(sl):
        pltpu.make_async_copy(rows[sl], agg_sh.at[pkb[sl].at[2]],
                              ssem[sl]).wait()

    fetch_idx(0, 0)
    fetch_idx(1, 1)
    fetch_rows(0, 0)
    fetch_rows(1, 1)

    def it(g, _):
        k0 = 2 * g
        process(0, k0)
        drain(0)
        fetch_idx(0, k0 + 2)
        fetch_rows(0, k0 + 2)
        process(1, k0 + 1)
        drain(1)
        fetch_idx(1, k0 + 3)
        fetch_rows(1, k0 + 3)
        return 0
    lax.fori_loop(0, CHT // 2, it, 0)
    for sl in (0, 1):
        pltpu.make_async_copy(hrows_hbm.at[pkb[sl].at[0]], rows[sl],
                              gsem[sl]).wait()
        pltpu.make_async_copy(norm_hbm.at[pkb[sl].at[1]], nrm[sl],
                              nsem[sl]).wait()

    plsc.subcore_barrier()
    pltpu.sync_copy(agg_sh.at[pl.ds(s * ART, ART)],
                    aggp_hbm.at[c, pl.ds(s * ART, ART)])


def _norm_body(cnt_ref, norm_ref):
    cnt = cnt_ref[0] + cnt_ref[1]
    norm_ref[...] = 1.0 / jnp.maximum(cnt, 1.0)


_tc_norm = pl.pallas_call(
    _norm_body,
    out_shape=jax.ShapeDtypeStruct((C // 128, 128), f32),
)


def _hmat_body(x_ref, w_ref, o_ref):
    o_ref[...] = jnp.dot(x_ref[...], w_ref[0], preferred_element_type=f32)


_tc_hmat = pl.pallas_call(
    _hmat_body,
    grid=(R,),
    in_specs=[pl.BlockSpec((NR, D), lambda r: (0, 0)),
              pl.BlockSpec((1, D, D), lambda r: (r, 0, 0))],
    out_specs=pl.BlockSpec((NR, D), lambda r: (r, 0)),
    out_shape=jax.ShapeDtypeStruct((R * NR, D), f32),
)


def _combine_body(x_ref, p0_ref, p1_ref, root_ref, b_ref, h_ref):
    h = p0_ref[...] + p1_ref[...] + b_ref[...]
    h = h + jnp.dot(x_ref[...], root_ref[...], preferred_element_type=f32)
    h_ref[...] = jnp.maximum(h, 0.0)


_tc_combine = pl.pallas_call(
    _combine_body,
    grid=(NR // 128,),
    in_specs=[pl.BlockSpec((128, D), lambda i: (i, 0)),
              pl.BlockSpec((128, D), lambda i: (i, 0)),
              pl.BlockSpec((128, D), lambda i: (i, 0)),
              pl.BlockSpec((D, D), lambda i: (0, 0)),
              pl.BlockSpec((1, D), lambda i: (0, 0))],
    out_specs=pl.BlockSpec((128, D), lambda i: (i, 0)),
    out_shape=jax.ShapeDtypeStruct((NR, D), f32),
)


def _readout_body(p0_ref, p1_ref, h1_ref, root_ref, b_ref, awt_ref, bs_ref,
                  lw_ref, lb_ref, o_ref, h2_sc, s_sc):
    h2 = p0_ref[...] + p1_ref[...] + b_ref[...]
    h2 = h2 + jnp.dot(h1_ref[...], root_ref[...], preferred_element_type=f32)
    h2 = jnp.maximum(h2, 0.0)
    h2_sc[...] = h2

    neg_inf = jnp.float32(-jnp.inf)
    giota = lax.broadcasted_iota(i32, (G, 128), 0)
    col = lax.broadcasted_iota(i32, (1, 128), 1)

    def oh_mask(b):
        bsrow = bs_ref[pl.ds(b, 1), :]
        vmask = (b * 128 + col) < N
        return (giota == bsrow) & vmask

    def pass1(b, m):
        hblk = h2_sc[pl.ds(b * 128, 128), :]
        scb = lax.dot_general(awt_ref[...], hblk, (((1,), (1,)), ((), ())),
                              preferred_element_type=f32)
        s_sc[pl.ds(b, 1), :] = scb
        oh = oh_mask(b)
        vals = jnp.where(oh, jnp.broadcast_to(scb, (G, 128)), neg_inf)
        return jnp.maximum(m, jnp.max(vals, axis=1, keepdims=True))

    m = lax.fori_loop(0, NR // 128, pass1, jnp.full((G, 1), neg_inf, f32))
    m = jnp.where(jnp.isfinite(m), m, 0.0)

    def pass2(b, carry):
        ssum, ge = carry
        scb = s_sc[pl.ds(b, 1), :]
        oh = oh_mask(b)
        mrow = jnp.sum(jnp.where(oh, jnp.broadcast_to(m, (G, 128)), 0.0),
                       axis=0, keepdims=True)
        erow = jnp.where((b * 128 + col) < N, jnp.exp(scb - mrow), 0.0)
        ohw = jnp.where(oh, jnp.broadcast_to(erow, (G, 128)), 0.0)
        hblk = h2_sc[pl.ds(b * 128, 128), :]
        return (ssum + jnp.sum(ohw, axis=1, keepdims=True),
                ge + jnp.dot(ohw, hblk, preferred_element_type=f32))

    ssum, ge = lax.fori_loop(0, NR // 128, pass2,
                             (jnp.zeros((G, 1), f32), jnp.zeros((G, D), f32)))
    ge = ge / jnp.where(ssum > 0.0, ssum, 1.0)
    rtu = jnp.dot(ge, lw_ref[...], preferred_element_type=f32) + lb_ref[...]
    o_ref[...] = jax.nn.sigmoid(rtu)


_tc_readout = pl.pallas_call(
    _readout_body,
    out_shape=jax.ShapeDtypeStruct((G, 1), f32),
    scratch_shapes=[pltpu.VMEM((NR, D), f32),
                    pltpu.VMEM((NR // 128, 128), f32)],
)


def kernel(nodeTypes, edge_index, edge_attr, bs, emb, W1, root1, b1,
           W2, root2, b2, att_w, lin_w, lin_b):
    src = edge_index[0]
    dst = edge_index[1]
    nt_p = jnp.pad(nodeTypes.astype(i32), (0, NP - N))
    pad_i = jnp.arange(EP - E, dtype=i32)
    src_p = jnp.concatenate([src.astype(i32), pad_i % N])
    dst_p = jnp.concatenate([dst.astype(i32), N + pad_i % (NR - N)])
    et_p = jnp.concatenate([edge_attr.astype(i32), pad_i % R])
    epk = jnp.stack([src_p, dst_p, et_p]).reshape(3, NCH, 128)
    epk = jnp.pad(epk.transpose(1, 0, 2), ((0, 4), (0, 0), (0, 0)))
    bs2 = jnp.pad(bs.astype(i32), (0, NR - N)).reshape(NR // 128, 128)

    x_np = _sc_prep_x(nt_p, emb)
    pk, cntp = _sc_prep(epk)
    norm = _tc_norm(cntp.reshape(2, C // 128, 128)).reshape(C)
    h1rows = _tc_hmat(x_np, W1)
    aggp1 = _sc_layer(h1rows, pk, norm)
    h1 = _tc_combine(x_np, aggp1[0], aggp1[1], root1, b1.reshape(1, D))
    aggp2 = _sc_layer(_tc_hmat(h1, W2), pk, norm)
    out = _tc_readout(aggp2[0], aggp2[1], h1, root2, b2.reshape(1, D),
                      att_w.reshape(D, 1).T, bs2, lin_w, lin_b.reshape(1, 1))
    return out
